# Initial kernel scaffold; baseline (speedup 1.0000x reference)
import jax
import jax.numpy as jnp
from jax.experimental import pallas as pl


def kernel(x, edge_indexes, edge_types, W1, root1, b1, W2, root2, b2):
    raise NotImplementedError("write your pallas kernel here")


# speedup vs baseline: 8.3860x; 8.3860x over previous
import functools

import jax
import jax.numpy as jnp
from jax import lax
from jax.experimental import pallas as pl
from jax.experimental.pallas import tpu as pltpu
from jax.experimental.pallas import tpu_sc as plsc

N = 10000
E = 320000
D = 128
R = 8
NR = N * R

NC = 2
NS = 16
NW = NC * NS
EPW = E // NW
K = 80
CHUNKS = EPW // K
EPT = E // NS
CHUNKS_AGG = EPT // K
NPAD = 10240
ROWS_PT = NPAD // NS
BINS_PT = NR // NS

_mesh = plsc.VectorSubcoreMesh(core_axis_name="c", subcore_axis_name="s")
_mesh1 = plsc.VectorSubcoreMesh(core_axis_name="c", subcore_axis_name="s",
                                num_cores=1)
_sc_params = pltpu.CompilerParams(needs_layout_passes=False)


@functools.partial(
    pl.kernel,
    out_type=jax.ShapeDtypeStruct((NC * NR,), jnp.float32),
    mesh=_mesh,
    scratch_types=[
        pltpu.VMEM((K,), jnp.int32),
        pltpu.VMEM((K,), jnp.int32),
        pltpu.VMEM((K,), jnp.int32),
        pltpu.VMEM((K,), jnp.float32),
        pltpu.VMEM((5008,), jnp.float32),
        pltpu.VMEM_SHARED((NR,), jnp.float32),
    ],
    compiler_params=_sc_params,
)
def _count(dst_hbm, typ_hbm, cnt_out, dst_v, typ_v, bin_v, one_v, stage_v,
           shared_cnt):
    c = lax.axis_index("c")
    s = lax.axis_index("s")
    wid = s * NC + c
    for j in range(K // 16):
        one_v[pl.ds(j * 16, 16)] = jnp.full((16,), 1.0, jnp.float32)

    def zloop(i, carry):
        stage_v[pl.ds(i * 16, 16)] = jnp.zeros((16,), jnp.float32)
        return carry

    lax.fori_loop(0, 5008 // 16, zloop, 0)
    pltpu.sync_copy(stage_v.at[pl.ds(0, BINS_PT)],
                    shared_cnt.at[pl.ds(s * BINS_PT, BINS_PT)])
    plsc.subcore_barrier()

    def chunk(i, carry):
        base = wid * EPW + i * K
        pltpu.sync_copy(dst_hbm.at[pl.ds(base, K)], dst_v)
        pltpu.sync_copy(typ_hbm.at[pl.ds(base, K)], typ_v)
        for j in range(K // 16):
            sl = pl.ds(j * 16, 16)
            bin_v[sl] = dst_v[sl] * R + typ_v[sl]
        pltpu.sync_copy(one_v, shared_cnt.at[bin_v], add=True)
        return carry

    lax.fori_loop(0, CHUNKS, chunk, 0)
    plsc.subcore_barrier()
    pltpu.sync_copy(shared_cnt.at[pl.ds(s * BINS_PT, BINS_PT)],
                    stage_v.at[pl.ds(0, BINS_PT)])
    pltpu.sync_copy(stage_v.at[pl.ds(0, BINS_PT)],
                    cnt_out.at[pl.ds(c * NR + s * BINS_PT, BINS_PT)])


@functools.partial(
    pl.kernel,
    out_type=jax.ShapeDtypeStruct((NPAD, D), jnp.float32),
    mesh=_mesh1,
    scratch_types=[
        pltpu.VMEM((K,), jnp.int32),
        pltpu.VMEM((K,), jnp.int32),
        pltpu.VMEM((K,), jnp.int32),
        pltpu.VMEM((K,), jnp.int32),
        pltpu.VMEM((K,), jnp.int32),
        pltpu.VMEM((K,), jnp.float32),
        pltpu.VMEM((5008,), jnp.float32),
        pltpu.VMEM((K, D), jnp.float32),
        pltpu.VMEM_SHARED((NR,), jnp.float32),
        pltpu.VMEM_SHARED((NPAD, D), jnp.float32),
        pltpu.SemaphoreType.DMA,
    ],
    compiler_params=_sc_params,
)
def _agg(xall_hbm, src_hbm, dst_hbm, typ_hbm, inv_hbm, parts_out,
         src_v, dst_v, typ_v, gidx_v, bin_v, scale_v, stage_v, rows_v,
         shared_inv, shared_acc, sem):
    s = lax.axis_index("s")
    pltpu.sync_copy(inv_hbm.at[pl.ds(s * BINS_PT, BINS_PT)],
                    stage_v.at[pl.ds(0, BINS_PT)])
    pltpu.sync_copy(stage_v.at[pl.ds(0, BINS_PT)],
                    shared_inv.at[pl.ds(s * BINS_PT, BINS_PT)])

    def zloop(i, carry):
        for q in range(D // 16):
            rows_v[i, pl.ds(q * 16, 16)] = jnp.zeros((16,), jnp.float32)
        return carry

    lax.fori_loop(0, K, zloop, 0)
    for k in range(ROWS_PT // K):
        pltpu.sync_copy(rows_v,
                        shared_acc.at[pl.ds(s * ROWS_PT + k * K, K)])
    plsc.subcore_barrier()

    def chunk(i, carry):
        base = s * EPT + i * K
        pltpu.sync_copy(src_hbm.at[pl.ds(base, K)], src_v)
        pltpu.sync_copy(dst_hbm.at[pl.ds(base, K)], dst_v)
        pltpu.sync_copy(typ_hbm.at[pl.ds(base, K)], typ_v)
        for j in range(K // 16):
            sl = pl.ds(j * 16, 16)
            sv = src_v[sl]
            dv = dst_v[sl]
            tv = typ_v[sl]
            gidx_v[sl] = tv * N + sv
            bin_v[sl] = dv * R + tv
        cp = pltpu.async_copy(xall_hbm.at[gidx_v], rows_v, sem)
        pltpu.sync_copy(shared_inv.at[bin_v], scale_v)
        cp.wait()

        def edge(e, carry2):
            sc = plsc.load_gather(scale_v, [jnp.broadcast_to(e, (16,))])
            for q in range(D // 16):
                sl2 = pl.ds(q * 16, 16)
                rows_v[e, sl2] = rows_v[e, sl2] * sc
            return carry2

        lax.fori_loop(0, K, edge, 0)
        pltpu.sync_copy(rows_v, shared_acc.at[dst_v], add=True)
        return carry

    lax.fori_loop(0, CHUNKS_AGG, chunk, 0)
    plsc.subcore_barrier()
    for k in range(ROWS_PT // K):
        pltpu.sync_copy(shared_acc.at[pl.ds(s * ROWS_PT + k * K, K)], rows_v)
        pltpu.sync_copy(rows_v,
                        parts_out.at[pl.ds(s * ROWS_PT + k * K, K)])


def _inv_body(cnt_ref, inv_ref):
    cnt = cnt_ref[pl.ds(0, NR)] + cnt_ref[pl.ds(NR, NR)]
    inv_ref[...] = 1.0 / jnp.maximum(cnt, 1.0)


def _inv(cnt_part):
    return pl.pallas_call(
        _inv_body,
        out_shape=jax.ShapeDtypeStruct((NR,), jnp.float32),
    )(cnt_part)


NB = 10
BN = N // NB


def _mm1_body(x_ref, wc_ref, root_ref, b_ref, xall_ref, base_ref):
    xb = x_ref[...]
    xall_ref[...] = jnp.dot(xb, wc_ref[0], preferred_element_type=jnp.float32)

    @pl.when(pl.program_id(1) == 0)
    def _():
        base_ref[...] = (
            jnp.dot(xb, root_ref[...], preferred_element_type=jnp.float32)
            + b_ref[...])


def _mm2_body(base1_ref, p_ref, wc_ref, root_ref, b_ref, xall_ref, base_ref):
    hb = jnp.maximum(base1_ref[...] + p_ref[...], 0.0)
    xall_ref[...] = jnp.dot(hb, wc_ref[0], preferred_element_type=jnp.float32)

    @pl.when(pl.program_id(1) == 0)
    def _():
        base_ref[...] = (
            jnp.dot(hb, root_ref[...], preferred_element_type=jnp.float32)
            + b_ref[...])


_mm_out = [
    jax.ShapeDtypeStruct((R * N, D), jnp.float32),
    jax.ShapeDtypeStruct((N, D), jnp.float32),
]
_mm_out_specs = [
    pl.BlockSpec((BN, D), lambda i, r: (r * NB + i, 0)),
    pl.BlockSpec((BN, D), lambda i, r: (i, 0)),
]
_w_specs = [
    pl.BlockSpec((1, D, D), lambda i, r: (r, 0, 0)),
    pl.BlockSpec((D, D), lambda i, r: (0, 0)),
    pl.BlockSpec((1, D), lambda i, r: (0, 0)),
]


def _mm1(x, W, root, b):
    return pl.pallas_call(
        _mm1_body,
        grid=(NB, R),
        in_specs=[pl.BlockSpec((BN, D), lambda i, r: (i, 0))] + _w_specs,
        out_specs=_mm_out_specs,
        out_shape=_mm_out,
    )(x, W, root, b.reshape(1, D))


def _mm2(base1, parts, W, root, b):
    return pl.pallas_call(
        _mm2_body,
        grid=(NB, R),
        in_specs=[pl.BlockSpec((BN, D), lambda i, r: (i, 0)),
                  pl.BlockSpec((BN, D), lambda i, r: (i, 0))]
        + _w_specs,
        out_specs=_mm_out_specs,
        out_shape=_mm_out,
    )(base1, parts, W, root, b.reshape(1, D))


def _combine_body(base_ref, p_ref, out_ref):
    out_ref[...] = base_ref[...] + p_ref[...]


def _combine(base, parts):
    return pl.pallas_call(
        _combine_body,
        grid=(NB,),
        in_specs=[
            pl.BlockSpec((BN, D), lambda i: (i, 0)),
            pl.BlockSpec((BN, D), lambda i: (i, 0)),
        ],
        out_specs=pl.BlockSpec((BN, D), lambda i: (i, 0)),
        out_shape=jax.ShapeDtypeStruct((N, D), jnp.float32),
    )(base, parts)


def kernel(x, edge_indexes, edge_types, W1, root1, b1, W2, root2, b2):
    src = edge_indexes[0]
    dst = edge_indexes[1]
    cnt_part = _count(dst, edge_types)
    inv = _inv(cnt_part)

    xall1, base1 = _mm1(x, W1, root1, b1)
    parts1 = _agg(xall1, src, dst, edge_types, inv)
    xall2, base2 = _mm2(base1, parts1, W2, root2, b2)
    parts2 = _agg(xall2, src, dst, edge_types, inv)
    return _combine(base2, parts2)


# speedup vs baseline: 14.7284x; 1.7563x over previous
import functools

import jax
import jax.numpy as jnp
from jax import lax
from jax.experimental import pallas as pl
from jax.experimental.pallas import tpu as pltpu
from jax.experimental.pallas import tpu_sc as plsc

N = 10000
E = 320000
D = 128
R = 8
NR = N * R

NC = 2
NS = 16
NW = NC * NS
EPW = E // NW
K = 80
CHUNKS = EPW // K
EPT = E // NS
KA = 160
CHUNKS_AGG = EPT // KA
NPAD = 10240
ROWS_PT = NPAD // NS
BINS_PT = NR // NS

_mesh = plsc.VectorSubcoreMesh(core_axis_name="c", subcore_axis_name="s")
_mesh1 = plsc.VectorSubcoreMesh(core_axis_name="c", subcore_axis_name="s",
                                num_cores=1)
_sc_params = pltpu.CompilerParams(needs_layout_passes=False)


@functools.partial(
    pl.kernel,
    out_type=jax.ShapeDtypeStruct((NC * NR,), jnp.float32),
    mesh=_mesh,
    scratch_types=[
        pltpu.VMEM((K,), jnp.int32),
        pltpu.VMEM((K,), jnp.int32),
        pltpu.VMEM((K,), jnp.int32),
        pltpu.VMEM((K,), jnp.float32),
        pltpu.VMEM((5008,), jnp.float32),
        pltpu.VMEM_SHARED((NR,), jnp.float32),
    ],
    compiler_params=_sc_params,
)
def _count(dst_hbm, typ_hbm, cnt_out, dst_v, typ_v, bin_v, one_v, stage_v,
           shared_cnt):
    c = lax.axis_index("c")
    s = lax.axis_index("s")
    wid = s * NC + c
    for j in range(K // 16):
        one_v[pl.ds(j * 16, 16)] = jnp.full((16,), 1.0, jnp.float32)

    def zloop(i, carry):
        stage_v[pl.ds(i * 16, 16)] = jnp.zeros((16,), jnp.float32)
        return carry

    lax.fori_loop(0, 5008 // 16, zloop, 0)
    pltpu.sync_copy(stage_v.at[pl.ds(0, BINS_PT)],
                    shared_cnt.at[pl.ds(s * BINS_PT, BINS_PT)])
    plsc.subcore_barrier()

    def chunk(i, carry):
        base = wid * EPW + i * K
        pltpu.sync_copy(dst_hbm.at[pl.ds(base, K)], dst_v)
        pltpu.sync_copy(typ_hbm.at[pl.ds(base, K)], typ_v)
        for j in range(K // 16):
            sl = pl.ds(j * 16, 16)
            bin_v[sl] = dst_v[sl] * R + typ_v[sl]
        pltpu.sync_copy(one_v, shared_cnt.at[bin_v], add=True)
        return carry

    lax.fori_loop(0, CHUNKS, chunk, 0)
    plsc.subcore_barrier()
    pltpu.sync_copy(shared_cnt.at[pl.ds(s * BINS_PT, BINS_PT)],
                    stage_v.at[pl.ds(0, BINS_PT)])
    pltpu.sync_copy(stage_v.at[pl.ds(0, BINS_PT)],
                    cnt_out.at[pl.ds(c * NR + s * BINS_PT, BINS_PT)])


@functools.partial(
    pl.kernel,
    out_type=[
        jax.ShapeDtypeStruct((E,), jnp.int32),
        jax.ShapeDtypeStruct((E,), jnp.float32),
    ],
    mesh=_mesh,
    scratch_types=[
        pltpu.VMEM((K,), jnp.int32),
        pltpu.VMEM((K,), jnp.int32),
        pltpu.VMEM((K,), jnp.int32),
        pltpu.VMEM((K,), jnp.int32),
        pltpu.VMEM((K,), jnp.int32),
        pltpu.VMEM((K,), jnp.float32),
        pltpu.VMEM((5008,), jnp.float32),
        pltpu.VMEM_SHARED((NR,), jnp.float32),
    ],
    compiler_params=_sc_params,
)
def _scale(src_hbm, dst_hbm, typ_hbm, inv_hbm, gidx_out, scale_out,
           src_v, dst_v, typ_v, gidx_v, bin_v, scale_v, stage_v, shared_inv):
    c = lax.axis_index("c")
    s = lax.axis_index("s")
    wid = s * NC + c
    pltpu.sync_copy(inv_hbm.at[pl.ds(s * BINS_PT, BINS_PT)],
                    stage_v.at[pl.ds(0, BINS_PT)])
    pltpu.sync_copy(stage_v.at[pl.ds(0, BINS_PT)],
                    shared_inv.at[pl.ds(s * BINS_PT, BINS_PT)])
    plsc.subcore_barrier()

    def chunk(i, carry):
        base = wid * EPW + i * K
        pltpu.sync_copy(src_hbm.at[pl.ds(base, K)], src_v)
        pltpu.sync_copy(dst_hbm.at[pl.ds(base, K)], dst_v)
        pltpu.sync_copy(typ_hbm.at[pl.ds(base, K)], typ_v)
        for j in range(K // 16):
            sl = pl.ds(j * 16, 16)
            sv = src_v[sl]
            dv = dst_v[sl]
            tv = typ_v[sl]
            gidx_v[sl] = tv * N + sv
            bin_v[sl] = dv * R + tv
        pltpu.sync_copy(shared_inv.at[bin_v], scale_v)
        pltpu.sync_copy(gidx_v, gidx_out.at[pl.ds(base, K)])
        pltpu.sync_copy(scale_v, scale_out.at[pl.ds(base, K)])
        return carry

    lax.fori_loop(0, CHUNKS, chunk, 0)


@functools.partial(
    pl.kernel,
    out_type=jax.ShapeDtypeStruct((NPAD, D), jnp.float32),
    mesh=_mesh1,
    scratch_types=[
        pltpu.VMEM((KA,), jnp.int32),
        pltpu.VMEM((KA,), jnp.int32),
        pltpu.VMEM((KA,), jnp.int32),
        pltpu.VMEM((KA,), jnp.int32),
        pltpu.VMEM((KA,), jnp.float32),
        pltpu.VMEM((KA,), jnp.float32),
        pltpu.VMEM((KA, D), jnp.float32),
        pltpu.VMEM((KA, D), jnp.float32),
        pltpu.VMEM_SHARED((NPAD, D), jnp.float32),
        pltpu.SemaphoreType.DMA,
        pltpu.SemaphoreType.DMA,
        pltpu.SemaphoreType.DMA,
        pltpu.SemaphoreType.DMA,
    ],
    compiler_params=_sc_params,
)
def _agg(xall_hbm, gidx_hbm, dst_hbm, scale_hbm, parts_out,
         g0, g1, d0, d1, s0, s1, r0, r1, shared_acc,
         sem_g, sem_d, sem_s, sem_r):
    s = lax.axis_index("s")
    gb = (g0, g1)
    db = (d0, d1)
    sb = (s0, s1)
    rb = (r0, r1)

    def zloop(i, carry):
        for q in range(D // 16):
            r0[i, pl.ds(q * 16, 16)] = jnp.zeros((16,), jnp.float32)
        return carry

    lax.fori_loop(0, KA, zloop, 0)
    for k in range(ROWS_PT // KA):
        pltpu.sync_copy(r0, shared_acc.at[pl.ds(s * ROWS_PT + k * KA, KA)])
    plsc.subcore_barrier()

    def fire_idx(i, p):
        base = s * EPT + i * KA
        pltpu.async_copy(gidx_hbm.at[pl.ds(base, KA)], gb[p], sem_g)
        pltpu.async_copy(dst_hbm.at[pl.ds(base, KA)], db[p], sem_d)
        pltpu.async_copy(scale_hbm.at[pl.ds(base, KA)], sb[p], sem_s)

    def wait_idx(p):
        pltpu.make_async_copy(gidx_hbm.at[pl.ds(0, KA)], gb[p], sem_g).wait()
        pltpu.make_async_copy(dst_hbm.at[pl.ds(0, KA)], db[p], sem_d).wait()
        pltpu.make_async_copy(scale_hbm.at[pl.ds(0, KA)], sb[p], sem_s).wait()

    def step(i, x, fire_next):
        y = 1 - x
        pltpu.make_async_copy(xall_hbm.at[gb[x]], rb[x], sem_r).wait()
        if fire_next:
            wait_idx(y)
            pltpu.async_copy(xall_hbm.at[gb[y]], rb[y], sem_r)

        def edge(u, carry2):
            for t in range(4):
                e = u * 4 + t
                sc = plsc.load_gather(sb[x], [jnp.broadcast_to(e, (16,))])
                for q in range(D // 16):
                    sl2 = pl.ds(q * 16, 16)
                    rb[x][e, sl2] = rb[x][e, sl2] * sc
            return carry2

        lax.fori_loop(0, KA // 4, edge, 0)
        pltpu.sync_copy(rb[x], shared_acc.at[db[x]], add=True)
        if fire_next:
            @pl.when(i + 2 < CHUNKS_AGG)
            def _():
                fire_idx(i + 2, x)

    fire_idx(0, 0)
    wait_idx(0)
    pltpu.async_copy(xall_hbm.at[g0], r0, sem_r)
    fire_idx(1, 1)

    def pair(p, carry):
        step(2 * p, 0, True)
        step(2 * p + 1, 1, True)
        return carry

    lax.fori_loop(0, (CHUNKS_AGG - 1) // 2, pair, 0)
    step(CHUNKS_AGG - 1, (CHUNKS_AGG - 1) % 2, False)

    plsc.subcore_barrier()
    for k in range(ROWS_PT // KA):
        pltpu.sync_copy(shared_acc.at[pl.ds(s * ROWS_PT + k * KA, KA)], r0)
        pltpu.sync_copy(r0, parts_out.at[pl.ds(s * ROWS_PT + k * KA, KA)])


def _inv_body(cnt_ref, inv_ref):
    cnt = cnt_ref[pl.ds(0, NR)] + cnt_ref[pl.ds(NR, NR)]
    inv_ref[...] = 1.0 / jnp.maximum(cnt, 1.0)


def _inv(cnt_part):
    return pl.pallas_call(
        _inv_body,
        out_shape=jax.ShapeDtypeStruct((NR,), jnp.float32),
    )(cnt_part)


NB = 10
BN = N // NB


def _mm1_body(x_ref, wc_ref, root_ref, b_ref, xall_ref, base_ref):
    xb = x_ref[...]
    xall_ref[...] = jnp.dot(xb, wc_ref[0], preferred_element_type=jnp.float32)

    @pl.when(pl.program_id(1) == 0)
    def _():
        base_ref[...] = (
            jnp.dot(xb, root_ref[...], preferred_element_type=jnp.float32)
            + b_ref[...])


def _mm2_body(base1_ref, p_ref, wc_ref, root_ref, b_ref, xall_ref, base_ref):
    hb = jnp.maximum(base1_ref[...] + p_ref[...], 0.0)
    xall_ref[...] = jnp.dot(hb, wc_ref[0], preferred_element_type=jnp.float32)

    @pl.when(pl.program_id(1) == 0)
    def _():
        base_ref[...] = (
            jnp.dot(hb, root_ref[...], preferred_element_type=jnp.float32)
            + b_ref[...])


_mm_out = [
    jax.ShapeDtypeStruct((R * N, D), jnp.float32),
    jax.ShapeDtypeStruct((N, D), jnp.float32),
]
_mm_out_specs = [
    pl.BlockSpec((BN, D), lambda i, r: (r * NB + i, 0)),
    pl.BlockSpec((BN, D), lambda i, r: (i, 0)),
]
_w_specs = [
    pl.BlockSpec((1, D, D), lambda i, r: (r, 0, 0)),
    pl.BlockSpec((D, D), lambda i, r: (0, 0)),
    pl.BlockSpec((1, D), lambda i, r: (0, 0)),
]


def _mm1(x, W, root, b):
    return pl.pallas_call(
        _mm1_body,
        grid=(NB, R),
        in_specs=[pl.BlockSpec((BN, D), lambda i, r: (i, 0))] + _w_specs,
        out_specs=_mm_out_specs,
        out_shape=_mm_out,
    )(x, W, root, b.reshape(1, D))


def _mm2(base1, parts, W, root, b):
    return pl.pallas_call(
        _mm2_body,
        grid=(NB, R),
        in_specs=[pl.BlockSpec((BN, D), lambda i, r: (i, 0)),
                  pl.BlockSpec((BN, D), lambda i, r: (i, 0))]
        + _w_specs,
        out_specs=_mm_out_specs,
        out_shape=_mm_out,
    )(base1, parts, W, root, b.reshape(1, D))


def _combine_body(base_ref, p_ref, out_ref):
    out_ref[...] = base_ref[...] + p_ref[...]


def _combine(base, parts):
    return pl.pallas_call(
        _combine_body,
        grid=(NB,),
        in_specs=[
            pl.BlockSpec((BN, D), lambda i: (i, 0)),
            pl.BlockSpec((BN, D), lambda i: (i, 0)),
        ],
        out_specs=pl.BlockSpec((BN, D), lambda i: (i, 0)),
        out_shape=jax.ShapeDtypeStruct((N, D), jnp.float32),
    )(base, parts)


def kernel(x, edge_indexes, edge_types, W1, root1, b1, W2, root2, b2):
    src = edge_indexes[0]
    dst = edge_indexes[1]
    cnt_part = _count(dst, edge_types)
    inv = _inv(cnt_part)
    gidx, scale = _scale(src, dst, edge_types, inv)

    xall1, base1 = _mm1(x, W1, root1, b1)
    parts1 = _agg(xall1, gidx, dst, scale)
    xall2, base2 = _mm2(base1, parts1, W2, root2, b2)
    parts2 = _agg(xall2, gidx, dst, scale)
    return _combine(base2, parts2)


# speedup vs baseline: 18.5738x; 1.2611x over previous
import functools

import jax
import jax.numpy as jnp
from jax import lax
from jax.experimental import pallas as pl
from jax.experimental.pallas import tpu as pltpu
from jax.experimental.pallas import tpu_sc as plsc

N = 10000
E = 320000
D = 128
R = 8
NR = N * R

NC = 2
NS = 16
NW = NC * NS
EPW = E // NW
K = 400
CHUNKS = EPW // K
EPT = E // NS
KA = 160
CHUNKS_AGG = EPT // KA
NPAD = 10240
ROWS_PT = NPAD // NS
BINS_PT = NR // NS

_mesh = plsc.VectorSubcoreMesh(core_axis_name="c", subcore_axis_name="s")
_mesh1 = plsc.VectorSubcoreMesh(core_axis_name="c", subcore_axis_name="s",
                                num_cores=1)
_sc_params = pltpu.CompilerParams(needs_layout_passes=False)


@functools.partial(
    pl.kernel,
    out_type=jax.ShapeDtypeStruct((NC * NR,), jnp.float32),
    mesh=_mesh,
    scratch_types=[
        pltpu.VMEM((K,), jnp.int32),
        pltpu.VMEM((K,), jnp.int32),
        pltpu.VMEM((K,), jnp.int32),
        pltpu.VMEM((K,), jnp.float32),
        pltpu.VMEM((5008,), jnp.float32),
        pltpu.VMEM_SHARED((NR,), jnp.float32),
        pltpu.SemaphoreType.DMA,
        pltpu.SemaphoreType.DMA,
    ],
    compiler_params=_sc_params,
)
def _count(dst_hbm, typ_hbm, cnt_out, dst_v, typ_v, bin_v, one_v, stage_v,
           shared_cnt, sem_d, sem_t):
    c = lax.axis_index("c")
    s = lax.axis_index("s")
    wid = s * NC + c
    for j in range(K // 16):
        one_v[pl.ds(j * 16, 16)] = jnp.full((16,), 1.0, jnp.float32)

    def zloop(i, carry):
        stage_v[pl.ds(i * 16, 16)] = jnp.zeros((16,), jnp.float32)
        return carry

    lax.fori_loop(0, 5008 // 16, zloop, 0)
    pltpu.sync_copy(stage_v.at[pl.ds(0, BINS_PT)],
                    shared_cnt.at[pl.ds(s * BINS_PT, BINS_PT)])
    plsc.subcore_barrier()

    def fire_loads(i):
        base = wid * EPW + i * K
        pltpu.async_copy(dst_hbm.at[pl.ds(base, K)], dst_v, sem_d)
        pltpu.async_copy(typ_hbm.at[pl.ds(base, K)], typ_v, sem_t)

    fire_loads(0)

    def chunk(i, carry):
        pltpu.make_async_copy(dst_hbm.at[pl.ds(0, K)], dst_v, sem_d).wait()
        pltpu.make_async_copy(typ_hbm.at[pl.ds(0, K)], typ_v, sem_t).wait()
        for j in range(K // 16):
            sl = pl.ds(j * 16, 16)
            bin_v[sl] = dst_v[sl] * R + typ_v[sl]

        @pl.when(i + 1 < CHUNKS)
        def _():
            fire_loads(i + 1)

        pltpu.sync_copy(one_v, shared_cnt.at[bin_v], add=True)
        return carry

    lax.fori_loop(0, CHUNKS, chunk, 0)
    plsc.subcore_barrier()
    pltpu.sync_copy(shared_cnt.at[pl.ds(s * BINS_PT, BINS_PT)],
                    stage_v.at[pl.ds(0, BINS_PT)])
    pltpu.sync_copy(stage_v.at[pl.ds(0, BINS_PT)],
                    cnt_out.at[pl.ds(c * NR + s * BINS_PT, BINS_PT)])


@functools.partial(
    pl.kernel,
    out_type=[
        jax.ShapeDtypeStruct((E,), jnp.int32),
        jax.ShapeDtypeStruct((E,), jnp.float32),
    ],
    mesh=_mesh,
    scratch_types=[
        pltpu.VMEM((K,), jnp.int32),
        pltpu.VMEM((K,), jnp.int32),
        pltpu.VMEM((K,), jnp.int32),
        pltpu.VMEM((K,), jnp.int32),
        pltpu.VMEM((K,), jnp.int32),
        pltpu.VMEM((K,), jnp.float32),
        pltpu.VMEM((5008,), jnp.float32),
        pltpu.VMEM_SHARED((NR,), jnp.float32),
        pltpu.SemaphoreType.DMA,
        pltpu.SemaphoreType.DMA,
        pltpu.SemaphoreType.DMA,
        pltpu.SemaphoreType.DMA,
        pltpu.SemaphoreType.DMA,
    ],
    compiler_params=_sc_params,
)
def _scale(src_hbm, dst_hbm, typ_hbm, inv_hbm, gidx_out, scale_out,
           src_v, dst_v, typ_v, gidx_v, bin_v, scale_v, stage_v, shared_inv,
           sem_s, sem_d, sem_t, sem_og, sem_os):
    c = lax.axis_index("c")
    s = lax.axis_index("s")
    wid = s * NC + c
    pltpu.sync_copy(inv_hbm.at[pl.ds(s * BINS_PT, BINS_PT)],
                    stage_v.at[pl.ds(0, BINS_PT)])
    pltpu.sync_copy(stage_v.at[pl.ds(0, BINS_PT)],
                    shared_inv.at[pl.ds(s * BINS_PT, BINS_PT)])
    plsc.subcore_barrier()

    def fire_loads(i):
        base = wid * EPW + i * K
        pltpu.async_copy(src_hbm.at[pl.ds(base, K)], src_v, sem_s)
        pltpu.async_copy(dst_hbm.at[pl.ds(base, K)], dst_v, sem_d)
        pltpu.async_copy(typ_hbm.at[pl.ds(base, K)], typ_v, sem_t)

    fire_loads(0)

    def chunk(i, carry):
        base = wid * EPW + i * K
        pltpu.make_async_copy(src_hbm.at[pl.ds(0, K)], src_v, sem_s).wait()
        pltpu.make_async_copy(dst_hbm.at[pl.ds(0, K)], dst_v, sem_d).wait()
        pltpu.make_async_copy(typ_hbm.at[pl.ds(0, K)], typ_v, sem_t).wait()

        @pl.when(i > 0)
        def _():
            pltpu.make_async_copy(gidx_v, gidx_out.at[pl.ds(0, K)],
                                  sem_og).wait()
            pltpu.make_async_copy(scale_v, scale_out.at[pl.ds(0, K)],
                                  sem_os).wait()

        for j in range(K // 16):
            sl = pl.ds(j * 16, 16)
            sv = src_v[sl]
            dv = dst_v[sl]
            tv = typ_v[sl]
            gidx_v[sl] = tv * N + sv
            bin_v[sl] = dv * R + tv

        @pl.when(i + 1 < CHUNKS)
        def _():
            fire_loads(i + 1)

        pltpu.sync_copy(shared_inv.at[bin_v], scale_v)
        pltpu.async_copy(gidx_v, gidx_out.at[pl.ds(base, K)], sem_og)
        pltpu.async_copy(scale_v, scale_out.at[pl.ds(base, K)], sem_os)
        return carry

    lax.fori_loop(0, CHUNKS, chunk, 0)
    pltpu.make_async_copy(gidx_v, gidx_out.at[pl.ds(0, K)], sem_og).wait()
    pltpu.make_async_copy(scale_v, scale_out.at[pl.ds(0, K)], sem_os).wait()


@functools.partial(
    pl.kernel,
    out_type=jax.ShapeDtypeStruct((NPAD, D), jnp.float32),
    mesh=_mesh1,
    scratch_types=[
        pltpu.VMEM((KA,), jnp.int32),
        pltpu.VMEM((KA,), jnp.int32),
        pltpu.VMEM((KA,), jnp.int32),
        pltpu.VMEM((KA,), jnp.int32),
        pltpu.VMEM((KA,), jnp.float32),
        pltpu.VMEM((KA,), jnp.float32),
        pltpu.VMEM((KA, D), jnp.float32),
        pltpu.VMEM((KA, D), jnp.float32),
        pltpu.VMEM_SHARED((NPAD, D), jnp.float32),
        pltpu.SemaphoreType.DMA,
        pltpu.SemaphoreType.DMA,
        pltpu.SemaphoreType.DMA,
        pltpu.SemaphoreType.DMA,
    ],
    compiler_params=_sc_params,
)
def _agg(xall_hbm, gidx_hbm, dst_hbm, scale_hbm, parts_out,
         g0, g1, d0, d1, s0, s1, r0, r1, shared_acc,
         sem_g, sem_d, sem_s, sem_r):
    s = lax.axis_index("s")
    gb = (g0, g1)
    db = (d0, d1)
    sb = (s0, s1)
    rb = (r0, r1)

    def zloop(i, carry):
        for q in range(D // 16):
            r0[i, pl.ds(q * 16, 16)] = jnp.zeros((16,), jnp.float32)
        return carry

    lax.fori_loop(0, KA, zloop, 0)
    for k in range(ROWS_PT // KA):
        pltpu.sync_copy(r0, shared_acc.at[pl.ds(s * ROWS_PT + k * KA, KA)])
    plsc.subcore_barrier()

    def fire_idx(i, p):
        base = s * EPT + i * KA
        pltpu.async_copy(gidx_hbm.at[pl.ds(base, KA)], gb[p], sem_g)
        pltpu.async_copy(dst_hbm.at[pl.ds(base, KA)], db[p], sem_d)
        pltpu.async_copy(scale_hbm.at[pl.ds(base, KA)], sb[p], sem_s)

    def wait_idx(p):
        pltpu.make_async_copy(gidx_hbm.at[pl.ds(0, KA)], gb[p], sem_g).wait()
        pltpu.make_async_copy(dst_hbm.at[pl.ds(0, KA)], db[p], sem_d).wait()
        pltpu.make_async_copy(scale_hbm.at[pl.ds(0, KA)], sb[p], sem_s).wait()

    def step(i, x, fire_next):
        y = 1 - x
        pltpu.make_async_copy(xall_hbm.at[gb[x]], rb[x], sem_r).wait()
        if fire_next:
            wait_idx(y)
            pltpu.async_copy(xall_hbm.at[gb[y]], rb[y], sem_r)

        def edge(u, carry2):
            for t in range(4):
                e = u * 4 + t
                sc = plsc.load_gather(sb[x], [jnp.broadcast_to(e, (16,))])
                for q in range(D // 16):
                    sl2 = pl.ds(q * 16, 16)
                    rb[x][e, sl2] = rb[x][e, sl2] * sc
            return carry2

        lax.fori_loop(0, KA // 4, edge, 0)
        pltpu.sync_copy(rb[x], shared_acc.at[db[x]], add=True)
        if fire_next:
            @pl.when(i + 2 < CHUNKS_AGG)
            def _():
                fire_idx(i + 2, x)

    fire_idx(0, 0)
    wait_idx(0)
    pltpu.async_copy(xall_hbm.at[g0], r0, sem_r)
    fire_idx(1, 1)

    def pair(p, carry):
        step(2 * p, 0, True)
        step(2 * p + 1, 1, True)
        return carry

    lax.fori_loop(0, (CHUNKS_AGG - 1) // 2, pair, 0)
    step(CHUNKS_AGG - 1, (CHUNKS_AGG - 1) % 2, False)

    plsc.subcore_barrier()
    for k in range(ROWS_PT // KA):
        pltpu.sync_copy(shared_acc.at[pl.ds(s * ROWS_PT + k * KA, KA)], r0)
        pltpu.sync_copy(r0, parts_out.at[pl.ds(s * ROWS_PT + k * KA, KA)])


def _inv_body(cnt_ref, inv_ref):
    cnt = cnt_ref[pl.ds(0, NR)] + cnt_ref[pl.ds(NR, NR)]
    inv_ref[...] = 1.0 / jnp.maximum(cnt, 1.0)


def _inv(cnt_part):
    return pl.pallas_call(
        _inv_body,
        out_shape=jax.ShapeDtypeStruct((NR,), jnp.float32),
    )(cnt_part)


NB = 10
BN = N // NB


def _mm1_body(x_ref, wc_ref, root_ref, b_ref, xall_ref, base_ref):
    xb = x_ref[...]
    xall_ref[...] = jnp.dot(xb, wc_ref[0], preferred_element_type=jnp.float32)

    @pl.when(pl.program_id(1) == 0)
    def _():
        base_ref[...] = (
            jnp.dot(xb, root_ref[...], preferred_element_type=jnp.float32)
            + b_ref[...])


def _mm2_body(base1_ref, p_ref, wc_ref, root_ref, b_ref, xall_ref, base_ref):
    hb = jnp.maximum(base1_ref[...] + p_ref[...], 0.0)
    xall_ref[...] = jnp.dot(hb, wc_ref[0], preferred_element_type=jnp.float32)

    @pl.when(pl.program_id(1) == 0)
    def _():
        base_ref[...] = (
            jnp.dot(hb, root_ref[...], preferred_element_type=jnp.float32)
            + b_ref[...])


_mm_out = [
    jax.ShapeDtypeStruct((R * N, D), jnp.float32),
    jax.ShapeDtypeStruct((N, D), jnp.float32),
]
_mm_out_specs = [
    pl.BlockSpec((BN, D), lambda i, r: (r * NB + i, 0)),
    pl.BlockSpec((BN, D), lambda i, r: (i, 0)),
]
_w_specs = [
    pl.BlockSpec((1, D, D), lambda i, r: (r, 0, 0)),
    pl.BlockSpec((D, D), lambda i, r: (0, 0)),
    pl.BlockSpec((1, D), lambda i, r: (0, 0)),
]


def _mm1(x, W, root, b):
    return pl.pallas_call(
        _mm1_body,
        grid=(NB, R),
        in_specs=[pl.BlockSpec((BN, D), lambda i, r: (i, 0))] + _w_specs,
        out_specs=_mm_out_specs,
        out_shape=_mm_out,
    )(x, W, root, b.reshape(1, D))


def _mm2(base1, parts, W, root, b):
    return pl.pallas_call(
        _mm2_body,
        grid=(NB, R),
        in_specs=[pl.BlockSpec((BN, D), lambda i, r: (i, 0)),
                  pl.BlockSpec((BN, D), lambda i, r: (i, 0))]
        + _w_specs,
        out_specs=_mm_out_specs,
        out_shape=_mm_out,
    )(base1, parts, W, root, b.reshape(1, D))


def _combine_body(base_ref, p_ref, out_ref):
    out_ref[...] = base_ref[...] + p_ref[...]


def _combine(base, parts):
    return pl.pallas_call(
        _combine_body,
        grid=(NB,),
        in_specs=[
            pl.BlockSpec((BN, D), lambda i: (i, 0)),
            pl.BlockSpec((BN, D), lambda i: (i, 0)),
        ],
        out_specs=pl.BlockSpec((BN, D), lambda i: (i, 0)),
        out_shape=jax.ShapeDtypeStruct((N, D), jnp.float32),
    )(base, parts)


def kernel(x, edge_indexes, edge_types, W1, root1, b1, W2, root2, b2):
    src = edge_indexes[0]
    dst = edge_indexes[1]
    cnt_part = _count(dst, edge_types)
    inv = _inv(cnt_part)
    gidx, scale = _scale(src, dst, edge_types, inv)

    xall1, base1 = _mm1(x, W1, root1, b1)
    parts1 = _agg(xall1, gidx, dst, scale)
    xall2, base2 = _mm2(base1, parts1, W2, root2, b2)
    parts2 = _agg(xall2, gidx, dst, scale)
    return _combine(base2, parts2)


# speedup vs baseline: 21.7421x; 1.1706x over previous
import functools

import jax
import jax.numpy as jnp
from jax import lax
from jax.experimental import pallas as pl
from jax.experimental.pallas import tpu as pltpu
from jax.experimental.pallas import tpu_sc as plsc

N = 10000
E = 320000
D = 128
R = 8
NR = N * R

NC = 2
NS = 16
NW = NC * NS
EPW = E // NW
K = 400
CHUNKS = EPW // K
EPT = E // NS
KA = 160
CHUNKS_AGG = EPT // KA
NPAD = 10240
ROWS_PT = NPAD // NS
BINS_PT = NR // NS

_mesh = plsc.VectorSubcoreMesh(core_axis_name="c", subcore_axis_name="s")
_mesh1 = plsc.VectorSubcoreMesh(core_axis_name="c", subcore_axis_name="s",
                                num_cores=1)
_sc_params = pltpu.CompilerParams(needs_layout_passes=False)


@functools.partial(
    pl.kernel,
    out_type=jax.ShapeDtypeStruct((NC * NR,), jnp.float32),
    mesh=_mesh,
    scratch_types=[
        pltpu.VMEM((K,), jnp.int32),
        pltpu.VMEM((K,), jnp.int32),
        pltpu.VMEM((K,), jnp.int32),
        pltpu.VMEM((K,), jnp.float32),
        pltpu.VMEM((5008,), jnp.float32),
        pltpu.VMEM_SHARED((NR,), jnp.float32),
        pltpu.SemaphoreType.DMA,
        pltpu.SemaphoreType.DMA,
    ],
    compiler_params=_sc_params,
)
def _count(dst_hbm, typ_hbm, cnt_out, dst_v, typ_v, bin_v, one_v, stage_v,
           shared_cnt, sem_d, sem_t):
    c = lax.axis_index("c")
    s = lax.axis_index("s")
    wid = s * NC + c
    for j in range(K // 16):
        one_v[pl.ds(j * 16, 16)] = jnp.full((16,), 1.0, jnp.float32)

    def zloop(i, carry):
        stage_v[pl.ds(i * 16, 16)] = jnp.zeros((16,), jnp.float32)
        return carry

    lax.fori_loop(0, 5008 // 16, zloop, 0)
    pltpu.sync_copy(stage_v.at[pl.ds(0, BINS_PT)],
                    shared_cnt.at[pl.ds(s * BINS_PT, BINS_PT)])
    plsc.subcore_barrier()

    def fire_loads(i):
        base = wid * EPW + i * K
        pltpu.async_copy(dst_hbm.at[pl.ds(base, K)], dst_v, sem_d)
        pltpu.async_copy(typ_hbm.at[pl.ds(base, K)], typ_v, sem_t)

    fire_loads(0)

    def chunk(i, carry):
        pltpu.make_async_copy(dst_hbm.at[pl.ds(0, K)], dst_v, sem_d).wait()
        pltpu.make_async_copy(typ_hbm.at[pl.ds(0, K)], typ_v, sem_t).wait()
        for j in range(K // 16):
            sl = pl.ds(j * 16, 16)
            bin_v[sl] = dst_v[sl] * R + typ_v[sl]

        @pl.when(i + 1 < CHUNKS)
        def _():
            fire_loads(i + 1)

        pltpu.sync_copy(one_v, shared_cnt.at[bin_v], add=True)
        return carry

    lax.fori_loop(0, CHUNKS, chunk, 0)
    plsc.subcore_barrier()
    pltpu.sync_copy(shared_cnt.at[pl.ds(s * BINS_PT, BINS_PT)],
                    stage_v.at[pl.ds(0, BINS_PT)])
    pltpu.sync_copy(stage_v.at[pl.ds(0, BINS_PT)],
                    cnt_out.at[pl.ds(c * NR + s * BINS_PT, BINS_PT)])


@functools.partial(
    pl.kernel,
    out_type=[
        jax.ShapeDtypeStruct((E,), jnp.int32),
        jax.ShapeDtypeStruct((E,), jnp.float32),
    ],
    mesh=_mesh,
    scratch_types=[
        pltpu.VMEM((K,), jnp.int32),
        pltpu.VMEM((K,), jnp.int32),
        pltpu.VMEM((K,), jnp.int32),
        pltpu.VMEM((K,), jnp.int32),
        pltpu.VMEM((K,), jnp.int32),
        pltpu.VMEM((K,), jnp.float32),
        pltpu.VMEM((5008,), jnp.float32),
        pltpu.VMEM_SHARED((NR,), jnp.float32),
        pltpu.SemaphoreType.DMA,
        pltpu.SemaphoreType.DMA,
        pltpu.SemaphoreType.DMA,
        pltpu.SemaphoreType.DMA,
        pltpu.SemaphoreType.DMA,
    ],
    compiler_params=_sc_params,
)
def _scale(src_hbm, dst_hbm, typ_hbm, inv_hbm, gidx_out, scale_out,
           src_v, dst_v, typ_v, gidx_v, bin_v, scale_v, stage_v, shared_inv,
           sem_s, sem_d, sem_t, sem_og, sem_os):
    c = lax.axis_index("c")
    s = lax.axis_index("s")
    wid = s * NC + c
    pltpu.sync_copy(inv_hbm.at[pl.ds(s * BINS_PT, BINS_PT)],
                    stage_v.at[pl.ds(0, BINS_PT)])
    pltpu.sync_copy(stage_v.at[pl.ds(0, BINS_PT)],
                    shared_inv.at[pl.ds(s * BINS_PT, BINS_PT)])
    plsc.subcore_barrier()

    def fire_loads(i):
        base = wid * EPW + i * K
        pltpu.async_copy(src_hbm.at[pl.ds(base, K)], src_v, sem_s)
        pltpu.async_copy(dst_hbm.at[pl.ds(base, K)], dst_v, sem_d)
        pltpu.async_copy(typ_hbm.at[pl.ds(base, K)], typ_v, sem_t)

    fire_loads(0)

    def chunk(i, carry):
        base = wid * EPW + i * K
        pltpu.make_async_copy(src_hbm.at[pl.ds(0, K)], src_v, sem_s).wait()
        pltpu.make_async_copy(dst_hbm.at[pl.ds(0, K)], dst_v, sem_d).wait()
        pltpu.make_async_copy(typ_hbm.at[pl.ds(0, K)], typ_v, sem_t).wait()

        @pl.when(i > 0)
        def _():
            pltpu.make_async_copy(gidx_v, gidx_out.at[pl.ds(0, K)],
                                  sem_og).wait()
            pltpu.make_async_copy(scale_v, scale_out.at[pl.ds(0, K)],
                                  sem_os).wait()

        for j in range(K // 16):
            sl = pl.ds(j * 16, 16)
            sv = src_v[sl]
            dv = dst_v[sl]
            tv = typ_v[sl]
            gidx_v[sl] = tv * N + sv
            bin_v[sl] = dv * R + tv

        @pl.when(i + 1 < CHUNKS)
        def _():
            fire_loads(i + 1)

        pltpu.sync_copy(shared_inv.at[bin_v], scale_v)
        pltpu.async_copy(gidx_v, gidx_out.at[pl.ds(base, K)], sem_og)
        pltpu.async_copy(scale_v, scale_out.at[pl.ds(base, K)], sem_os)
        return carry

    lax.fori_loop(0, CHUNKS, chunk, 0)
    pltpu.make_async_copy(gidx_v, gidx_out.at[pl.ds(0, K)], sem_og).wait()
    pltpu.make_async_copy(scale_v, scale_out.at[pl.ds(0, K)], sem_os).wait()


@functools.partial(
    pl.kernel,
    out_type=jax.ShapeDtypeStruct((NPAD, D), jnp.float32),
    mesh=_mesh1,
    scratch_types=[
        pltpu.VMEM((KA,), jnp.int32),
        pltpu.VMEM((KA,), jnp.int32),
        pltpu.VMEM((KA,), jnp.int32),
        pltpu.VMEM((KA,), jnp.int32),
        pltpu.VMEM((KA,), jnp.float32),
        pltpu.VMEM((KA,), jnp.float32),
        pltpu.VMEM((KA,), jnp.int32),
        pltpu.VMEM((KA,), jnp.int32),
        pltpu.VMEM((KA, D), jnp.float32),
        pltpu.VMEM((KA, D), jnp.float32),
        pltpu.VMEM_SHARED((NPAD, D), jnp.float32),
        pltpu.SemaphoreType.DMA,
        pltpu.SemaphoreType.DMA,
        pltpu.SemaphoreType.DMA,
        pltpu.SemaphoreType.DMA,
        pltpu.SemaphoreType.DMA,
    ],
    compiler_params=_sc_params,
)
def _agg(xall_hbm, gidx_hbm, dst_hbm, scale_hbm, parts_out,
         g0, g1, d0, d1, s0, s1, dp0, dp1, r0, r1, shared_acc,
         sem_g, sem_d, sem_s, sem_r, sem_w):
    s = lax.axis_index("s")
    gb = (g0, g1)
    db = (d0, d1)
    sb = (s0, s1)
    dpb = (dp0, dp1)
    rb = (r0, r1)

    def zloop(i, carry):
        for q in range(D // 16):
            r0[i, pl.ds(q * 16, 16)] = jnp.zeros((16,), jnp.float32)
        return carry

    lax.fori_loop(0, KA, zloop, 0)
    for k in range(ROWS_PT // KA):
        pltpu.sync_copy(r0, shared_acc.at[pl.ds(s * ROWS_PT + k * KA, KA)])
    plsc.subcore_barrier()

    def fire_idx(i, p):
        base = s * EPT + i * KA
        pltpu.async_copy(gidx_hbm.at[pl.ds(base, KA)], gb[p], sem_g)
        pltpu.async_copy(dst_hbm.at[pl.ds(base, KA)], db[p], sem_d)
        pltpu.async_copy(scale_hbm.at[pl.ds(base, KA)], sb[p], sem_s)

    def wait_idx(p):
        pltpu.make_async_copy(gidx_hbm.at[pl.ds(0, KA)], gb[p], sem_g).wait()
        pltpu.make_async_copy(dst_hbm.at[pl.ds(0, KA)], db[p], sem_d).wait()
        pltpu.make_async_copy(scale_hbm.at[pl.ds(0, KA)], sb[p], sem_s).wait()

    def step(i, x, fire_next):
        y = 1 - x
        pltpu.make_async_copy(xall_hbm.at[gb[x]], rb[x], sem_r).wait()

        @pl.when(i >= 1)
        def _():
            pltpu.make_async_copy(rb[y], shared_acc.at[dpb[y]], sem_w).wait()

        for j in range(KA // 16):
            sl = pl.ds(j * 16, 16)
            dpb[x][sl] = db[x][sl]
        if fire_next:
            wait_idx(y)
            pltpu.async_copy(xall_hbm.at[gb[y]], rb[y], sem_r)

        def edge(u, carry2):
            for t in range(4):
                e = u * 4 + t
                sc = plsc.load_gather(sb[x], [jnp.broadcast_to(e, (16,))])
                for q in range(D // 16):
                    sl2 = pl.ds(q * 16, 16)
                    rb[x][e, sl2] = rb[x][e, sl2] * sc
            return carry2

        lax.fori_loop(0, KA // 4, edge, 0)
        pltpu.async_copy(rb[x], shared_acc.at[dpb[x]], sem_w, add=True)
        if fire_next:
            @pl.when(i + 2 < CHUNKS_AGG)
            def _():
                fire_idx(i + 2, x)

    fire_idx(0, 0)
    wait_idx(0)
    pltpu.async_copy(xall_hbm.at[g0], r0, sem_r)
    fire_idx(1, 1)

    def pair(p, carry):
        step(2 * p, 0, True)
        step(2 * p + 1, 1, True)
        return carry

    lax.fori_loop(0, (CHUNKS_AGG - 1) // 2, pair, 0)
    step(CHUNKS_AGG - 1, (CHUNKS_AGG - 1) % 2, False)
    pltpu.make_async_copy(
        rb[(CHUNKS_AGG - 1) % 2],
        shared_acc.at[dpb[(CHUNKS_AGG - 1) % 2]], sem_w).wait()

    plsc.subcore_barrier()
    for k in range(ROWS_PT // KA):
        pltpu.sync_copy(shared_acc.at[pl.ds(s * ROWS_PT + k * KA, KA)], r0)
        pltpu.sync_copy(r0, parts_out.at[pl.ds(s * ROWS_PT + k * KA, KA)])


def _inv_body(cnt_ref, inv_ref):
    cnt = cnt_ref[pl.ds(0, NR)] + cnt_ref[pl.ds(NR, NR)]
    inv_ref[...] = 1.0 / jnp.maximum(cnt, 1.0)


def _inv(cnt_part):
    return pl.pallas_call(
        _inv_body,
        out_shape=jax.ShapeDtypeStruct((NR,), jnp.float32),
    )(cnt_part)


NB = 10
BN = N // NB


def _mm1_body(x_ref, wc_ref, root_ref, b_ref, xall_ref, base_ref):
    xb = x_ref[...]
    xall_ref[...] = jnp.dot(xb, wc_ref[0], preferred_element_type=jnp.float32)

    @pl.when(pl.program_id(1) == 0)
    def _():
        base_ref[...] = (
            jnp.dot(xb, root_ref[...], preferred_element_type=jnp.float32)
            + b_ref[...])


def _mm2_body(base1_ref, p_ref, wc_ref, root_ref, b_ref, xall_ref, base_ref):
    hb = jnp.maximum(base1_ref[...] + p_ref[...], 0.0)
    xall_ref[...] = jnp.dot(hb, wc_ref[0], preferred_element_type=jnp.float32)

    @pl.when(pl.program_id(1) == 0)
    def _():
        base_ref[...] = (
            jnp.dot(hb, root_ref[...], preferred_element_type=jnp.float32)
            + b_ref[...])


_mm_out = [
    jax.ShapeDtypeStruct((R * N, D), jnp.float32),
    jax.ShapeDtypeStruct((N, D), jnp.float32),
]
_mm_out_specs = [
    pl.BlockSpec((BN, D), lambda i, r: (r * NB + i, 0)),
    pl.BlockSpec((BN, D), lambda i, r: (i, 0)),
]
_w_specs = [
    pl.BlockSpec((1, D, D), lambda i, r: (r, 0, 0)),
    pl.BlockSpec((D, D), lambda i, r: (0, 0)),
    pl.BlockSpec((1, D), lambda i, r: (0, 0)),
]


def _mm1(x, W, root, b):
    return pl.pallas_call(
        _mm1_body,
        grid=(NB, R),
        in_specs=[pl.BlockSpec((BN, D), lambda i, r: (i, 0))] + _w_specs,
        out_specs=_mm_out_specs,
        out_shape=_mm_out,
    )(x, W, root, b.reshape(1, D))


def _mm2(base1, parts, W, root, b):
    return pl.pallas_call(
        _mm2_body,
        grid=(NB, R),
        in_specs=[pl.BlockSpec((BN, D), lambda i, r: (i, 0)),
                  pl.BlockSpec((BN, D), lambda i, r: (i, 0))]
        + _w_specs,
        out_specs=_mm_out_specs,
        out_shape=_mm_out,
    )(base1, parts, W, root, b.reshape(1, D))


def _combine_body(base_ref, p_ref, out_ref):
    out_ref[...] = base_ref[...] + p_ref[...]


def _combine(base, parts):
    return pl.pallas_call(
        _combine_body,
        grid=(NB,),
        in_specs=[
            pl.BlockSpec((BN, D), lambda i: (i, 0)),
            pl.BlockSpec((BN, D), lambda i: (i, 0)),
        ],
        out_specs=pl.BlockSpec((BN, D), lambda i: (i, 0)),
        out_shape=jax.ShapeDtypeStruct((N, D), jnp.float32),
    )(base, parts)


def kernel(x, edge_indexes, edge_types, W1, root1, b1, W2, root2, b2):
    src = edge_indexes[0]
    dst = edge_indexes[1]
    cnt_part = _count(dst, edge_types)
    inv = _inv(cnt_part)
    gidx, scale = _scale(src, dst, edge_types, inv)

    xall1, base1 = _mm1(x, W1, root1, b1)
    parts1 = _agg(xall1, gidx, dst, scale)
    xall2, base2 = _mm2(base1, parts1, W2, root2, b2)
    parts2 = _agg(xall2, gidx, dst, scale)
    return _combine(base2, parts2)


# speedup vs baseline: 21.8107x; 1.0032x over previous
import functools

import jax
import jax.numpy as jnp
from jax import lax
from jax.experimental import pallas as pl
from jax.experimental.pallas import tpu as pltpu
from jax.experimental.pallas import tpu_sc as plsc

N = 10000
E = 320000
D = 128
R = 8
NR = N * R

NC = 2
NS = 16
NW = NC * NS
EPW = E // NW
K = 400
CHUNKS = EPW // K
EPT = E // NS
KA = 160
CHUNKS_AGG = EPT // KA
NPAD = 10240
ROWS_PT = NPAD // NS
BINS_PT = NR // NS

_mesh = plsc.VectorSubcoreMesh(core_axis_name="c", subcore_axis_name="s")
_mesh1 = plsc.VectorSubcoreMesh(core_axis_name="c", subcore_axis_name="s",
                                num_cores=1)
_sc_params = pltpu.CompilerParams(needs_layout_passes=False)


@functools.partial(
    pl.kernel,
    out_type=jax.ShapeDtypeStruct((NC * NR,), jnp.float32),
    mesh=_mesh,
    scratch_types=[
        pltpu.VMEM((K,), jnp.int32),
        pltpu.VMEM((K,), jnp.int32),
        pltpu.VMEM((K,), jnp.int32),
        pltpu.VMEM((K,), jnp.float32),
        pltpu.VMEM((5008,), jnp.float32),
        pltpu.VMEM_SHARED((NR,), jnp.float32),
        pltpu.SemaphoreType.DMA,
        pltpu.SemaphoreType.DMA,
    ],
    compiler_params=_sc_params,
)
def _count(dst_hbm, typ_hbm, cnt_out, dst_v, typ_v, bin_v, one_v, stage_v,
           shared_cnt, sem_d, sem_t):
    c = lax.axis_index("c")
    s = lax.axis_index("s")
    wid = s * NC + c
    for j in range(K // 16):
        one_v[pl.ds(j * 16, 16)] = jnp.full((16,), 1.0, jnp.float32)

    def zloop(i, carry):
        stage_v[pl.ds(i * 16, 16)] = jnp.zeros((16,), jnp.float32)
        return carry

    lax.fori_loop(0, 5008 // 16, zloop, 0)
    pltpu.sync_copy(stage_v.at[pl.ds(0, BINS_PT)],
                    shared_cnt.at[pl.ds(s * BINS_PT, BINS_PT)])
    plsc.subcore_barrier()

    def fire_loads(i):
        base = wid * EPW + i * K
        pltpu.async_copy(dst_hbm.at[pl.ds(base, K)], dst_v, sem_d)
        pltpu.async_copy(typ_hbm.at[pl.ds(base, K)], typ_v, sem_t)

    fire_loads(0)

    def chunk(i, carry):
        pltpu.make_async_copy(dst_hbm.at[pl.ds(0, K)], dst_v, sem_d).wait()
        pltpu.make_async_copy(typ_hbm.at[pl.ds(0, K)], typ_v, sem_t).wait()
        for j in range(K // 16):
            sl = pl.ds(j * 16, 16)
            bin_v[sl] = dst_v[sl] * R + typ_v[sl]

        @pl.when(i + 1 < CHUNKS)
        def _():
            fire_loads(i + 1)

        pltpu.sync_copy(one_v, shared_cnt.at[bin_v], add=True)
        return carry

    lax.fori_loop(0, CHUNKS, chunk, 0)
    plsc.subcore_barrier()
    pltpu.sync_copy(shared_cnt.at[pl.ds(s * BINS_PT, BINS_PT)],
                    stage_v.at[pl.ds(0, BINS_PT)])
    pltpu.sync_copy(stage_v.at[pl.ds(0, BINS_PT)],
                    cnt_out.at[pl.ds(c * NR + s * BINS_PT, BINS_PT)])


@functools.partial(
    pl.kernel,
    out_type=[
        jax.ShapeDtypeStruct((E,), jnp.int32),
        jax.ShapeDtypeStruct((E,), jnp.float32),
    ],
    mesh=_mesh,
    scratch_types=[
        pltpu.VMEM((K,), jnp.int32),
        pltpu.VMEM((K,), jnp.int32),
        pltpu.VMEM((K,), jnp.int32),
        pltpu.VMEM((K,), jnp.int32),
        pltpu.VMEM((K,), jnp.int32),
        pltpu.VMEM((K,), jnp.float32),
        pltpu.VMEM((5008,), jnp.float32),
        pltpu.VMEM_SHARED((NR,), jnp.float32),
        pltpu.SemaphoreType.DMA,
        pltpu.SemaphoreType.DMA,
        pltpu.SemaphoreType.DMA,
        pltpu.SemaphoreType.DMA,
        pltpu.SemaphoreType.DMA,
    ],
    compiler_params=_sc_params,
)
def _scale(src_hbm, dst_hbm, typ_hbm, inv_hbm, gidx_out, scale_out,
           src_v, dst_v, typ_v, gidx_v, bin_v, scale_v, stage_v, shared_inv,
           sem_s, sem_d, sem_t, sem_og, sem_os):
    c = lax.axis_index("c")
    s = lax.axis_index("s")
    wid = s * NC + c
    pltpu.sync_copy(inv_hbm.at[pl.ds(s * BINS_PT, BINS_PT)],
                    stage_v.at[pl.ds(0, BINS_PT)])
    pltpu.sync_copy(stage_v.at[pl.ds(0, BINS_PT)],
                    shared_inv.at[pl.ds(s * BINS_PT, BINS_PT)])
    plsc.subcore_barrier()

    def fire_loads(i):
        base = wid * EPW + i * K
        pltpu.async_copy(src_hbm.at[pl.ds(base, K)], src_v, sem_s)
        pltpu.async_copy(dst_hbm.at[pl.ds(base, K)], dst_v, sem_d)
        pltpu.async_copy(typ_hbm.at[pl.ds(base, K)], typ_v, sem_t)

    fire_loads(0)

    def chunk(i, carry):
        base = wid * EPW + i * K
        pltpu.make_async_copy(src_hbm.at[pl.ds(0, K)], src_v, sem_s).wait()
        pltpu.make_async_copy(dst_hbm.at[pl.ds(0, K)], dst_v, sem_d).wait()
        pltpu.make_async_copy(typ_hbm.at[pl.ds(0, K)], typ_v, sem_t).wait()

        @pl.when(i > 0)
        def _():
            pltpu.make_async_copy(gidx_v, gidx_out.at[pl.ds(0, K)],
                                  sem_og).wait()
            pltpu.make_async_copy(scale_v, scale_out.at[pl.ds(0, K)],
                                  sem_os).wait()

        for j in range(K // 16):
            sl = pl.ds(j * 16, 16)
            sv = src_v[sl]
            dv = dst_v[sl]
            tv = typ_v[sl]
            gidx_v[sl] = tv * N + sv
            bin_v[sl] = dv * R + tv

        @pl.when(i + 1 < CHUNKS)
        def _():
            fire_loads(i + 1)

        pltpu.sync_copy(shared_inv.at[bin_v], scale_v)
        pltpu.async_copy(gidx_v, gidx_out.at[pl.ds(base, K)], sem_og)
        pltpu.async_copy(scale_v, scale_out.at[pl.ds(base, K)], sem_os)
        return carry

    lax.fori_loop(0, CHUNKS, chunk, 0)
    pltpu.make_async_copy(gidx_v, gidx_out.at[pl.ds(0, K)], sem_og).wait()
    pltpu.make_async_copy(scale_v, scale_out.at[pl.ds(0, K)], sem_os).wait()


@functools.partial(
    pl.kernel,
    out_type=jax.ShapeDtypeStruct((NPAD, D), jnp.float32),
    mesh=_mesh1,
    scratch_types=[
        pltpu.VMEM((KA,), jnp.int32),
        pltpu.VMEM((KA,), jnp.int32),
        pltpu.VMEM((KA,), jnp.int32),
        pltpu.VMEM((KA,), jnp.int32),
        pltpu.VMEM((KA,), jnp.float32),
        pltpu.VMEM((KA,), jnp.float32),
        pltpu.VMEM((KA,), jnp.int32),
        pltpu.VMEM((KA,), jnp.int32),
        pltpu.VMEM((KA, D), jnp.float32),
        pltpu.VMEM((KA, D), jnp.float32),
        pltpu.VMEM_SHARED((NPAD, D), jnp.float32),
        pltpu.SemaphoreType.DMA,
        pltpu.SemaphoreType.DMA,
        pltpu.SemaphoreType.DMA,
        pltpu.SemaphoreType.DMA,
        pltpu.SemaphoreType.DMA,
    ],
    compiler_params=_sc_params,
)
def _agg(xall_hbm, gidx_hbm, dst_hbm, scale_hbm, parts_out,
         g0, g1, d0, d1, s0, s1, dp0, dp1, r0, r1, shared_acc,
         sem_g, sem_d, sem_s, sem_r, sem_w):
    s = lax.axis_index("s")
    gb = (g0, g1)
    db = (d0, d1)
    sb = (s0, s1)
    dpb = (dp0, dp1)
    rb = (r0, r1)

    def zloop(i, carry):
        for q in range(D // 16):
            r0[i, pl.ds(q * 16, 16)] = jnp.zeros((16,), jnp.float32)
        return carry

    lax.fori_loop(0, KA, zloop, 0)
    for k in range(ROWS_PT // KA):
        pltpu.sync_copy(r0, shared_acc.at[pl.ds(s * ROWS_PT + k * KA, KA)])
    plsc.subcore_barrier()

    def fire_idx(i, p):
        base = s * EPT + i * KA
        pltpu.async_copy(gidx_hbm.at[pl.ds(base, KA)], gb[p], sem_g)
        pltpu.async_copy(dst_hbm.at[pl.ds(base, KA)], db[p], sem_d)
        pltpu.async_copy(scale_hbm.at[pl.ds(base, KA)], sb[p], sem_s)

    def wait_idx(p):
        pltpu.make_async_copy(gidx_hbm.at[pl.ds(0, KA)], gb[p], sem_g).wait()
        pltpu.make_async_copy(dst_hbm.at[pl.ds(0, KA)], db[p], sem_d).wait()
        pltpu.make_async_copy(scale_hbm.at[pl.ds(0, KA)], sb[p], sem_s).wait()

    def step(i, x, fire_next):
        y = 1 - x
        pltpu.make_async_copy(xall_hbm.at[gb[x]], rb[x], sem_r).wait()

        @pl.when(i >= 1)
        def _():
            pltpu.make_async_copy(rb[y], shared_acc.at[dpb[y]], sem_w).wait()

        for j in range(KA // 16):
            sl = pl.ds(j * 16, 16)
            dpb[x][sl] = db[x][sl]
        if fire_next:
            wait_idx(y)
            pltpu.async_copy(xall_hbm.at[gb[y]], rb[y], sem_r)

        def edge(u, carry2):
            for t in range(8):
                e = u * 8 + t
                sc = plsc.load_gather(sb[x], [jnp.broadcast_to(e, (16,))])
                for q in range(D // 16):
                    sl2 = pl.ds(q * 16, 16)
                    rb[x][e, sl2] = rb[x][e, sl2] * sc
            return carry2

        lax.fori_loop(0, KA // 8, edge, 0)
        pltpu.async_copy(rb[x], shared_acc.at[dpb[x]], sem_w, add=True)
        if fire_next:
            @pl.when(i + 2 < CHUNKS_AGG)
            def _():
                fire_idx(i + 2, x)

    fire_idx(0, 0)
    wait_idx(0)
    pltpu.async_copy(xall_hbm.at[g0], r0, sem_r)
    fire_idx(1, 1)

    def pair(p, carry):
        step(2 * p, 0, True)
        step(2 * p + 1, 1, True)
        return carry

    lax.fori_loop(0, (CHUNKS_AGG - 1) // 2, pair, 0)
    step(CHUNKS_AGG - 1, (CHUNKS_AGG - 1) % 2, False)
    pltpu.make_async_copy(
        rb[(CHUNKS_AGG - 1) % 2],
        shared_acc.at[dpb[(CHUNKS_AGG - 1) % 2]], sem_w).wait()

    plsc.subcore_barrier()
    for k in range(ROWS_PT // KA):
        pltpu.sync_copy(shared_acc.at[pl.ds(s * ROWS_PT + k * KA, KA)], r0)
        pltpu.sync_copy(r0, parts_out.at[pl.ds(s * ROWS_PT + k * KA, KA)])


def _inv_body(cnt_ref, inv_ref):
    cnt = cnt_ref[pl.ds(0, NR)] + cnt_ref[pl.ds(NR, NR)]
    inv_ref[...] = 1.0 / jnp.maximum(cnt, 1.0)


def _inv(cnt_part):
    return pl.pallas_call(
        _inv_body,
        out_shape=jax.ShapeDtypeStruct((NR,), jnp.float32),
    )(cnt_part)


NB = 10
BN = N // NB


def _mm1_body(x_ref, wc_ref, root_ref, b_ref, xall_ref, base_ref):
    xb = x_ref[...]
    xall_ref[...] = jnp.dot(xb, wc_ref[0], preferred_element_type=jnp.float32)

    @pl.when(pl.program_id(1) == 0)
    def _():
        base_ref[...] = (
            jnp.dot(xb, root_ref[...], preferred_element_type=jnp.float32)
            + b_ref[...])


def _mm2_body(base1_ref, p_ref, wc_ref, root_ref, b_ref, xall_ref, base_ref):
    hb = jnp.maximum(base1_ref[...] + p_ref[...], 0.0)
    xall_ref[...] = jnp.dot(hb, wc_ref[0], preferred_element_type=jnp.float32)

    @pl.when(pl.program_id(1) == 0)
    def _():
        base_ref[...] = (
            jnp.dot(hb, root_ref[...], preferred_element_type=jnp.float32)
            + b_ref[...])


_mm_out = [
    jax.ShapeDtypeStruct((R * N, D), jnp.float32),
    jax.ShapeDtypeStruct((N, D), jnp.float32),
]
_mm_out_specs = [
    pl.BlockSpec((BN, D), lambda i, r: (r * NB + i, 0)),
    pl.BlockSpec((BN, D), lambda i, r: (i, 0)),
]
_w_specs = [
    pl.BlockSpec((1, D, D), lambda i, r: (r, 0, 0)),
    pl.BlockSpec((D, D), lambda i, r: (0, 0)),
    pl.BlockSpec((1, D), lambda i, r: (0, 0)),
]


def _mm1(x, W, root, b):
    return pl.pallas_call(
        _mm1_body,
        grid=(NB, R),
        in_specs=[pl.BlockSpec((BN, D), lambda i, r: (i, 0))] + _w_specs,
        out_specs=_mm_out_specs,
        out_shape=_mm_out,
    )(x, W, root, b.reshape(1, D))


def _mm2(base1, parts, W, root, b):
    return pl.pallas_call(
        _mm2_body,
        grid=(NB, R),
        in_specs=[pl.BlockSpec((BN, D), lambda i, r: (i, 0)),
                  pl.BlockSpec((BN, D), lambda i, r: (i, 0))]
        + _w_specs,
        out_specs=_mm_out_specs,
        out_shape=_mm_out,
    )(base1, parts, W, root, b.reshape(1, D))


def _combine_body(base_ref, p_ref, out_ref):
    out_ref[...] = base_ref[...] + p_ref[...]


def _combine(base, parts):
    return pl.pallas_call(
        _combine_body,
        grid=(NB,),
        in_specs=[
            pl.BlockSpec((BN, D), lambda i: (i, 0)),
            pl.BlockSpec((BN, D), lambda i: (i, 0)),
        ],
        out_specs=pl.BlockSpec((BN, D), lambda i: (i, 0)),
        out_shape=jax.ShapeDtypeStruct((N, D), jnp.float32),
    )(base, parts)


def kernel(x, edge_indexes, edge_types, W1, root1, b1, W2, root2, b2):
    src = edge_indexes[0]
    dst = edge_indexes[1]
    cnt_part = _count(dst, edge_types)
    inv = _inv(cnt_part)
    gidx, scale = _scale(src, dst, edge_types, inv)

    xall1, base1 = _mm1(x, W1, root1, b1)
    parts1 = _agg(xall1, gidx, dst, scale)
    xall2, base2 = _mm2(base1, parts1, W2, root2, b2)
    parts2 = _agg(xall2, gidx, dst, scale)
    return _combine(base2, parts2)
